# baseline (speedup 1.0000x reference)
import functools

import jax
import jax.numpy as jnp
from jax import lax
from jax.experimental import pallas as pl
from jax.experimental.pallas import tpu as pltpu
from jax.experimental.pallas import tpu_sc as plsc

NC = 2
NS = 16
NW = NC * NS

IDXW = 128
KSUB = 4
CHUNK = KSUB * IDXW


@functools.lru_cache(maxsize=None)
def _build(B, V, D):
    assert B % (NW * CHUNK) == 0
    rows_per_w = B // NW
    n_iter = rows_per_w // CHUNK
    idxrows_per_w = rows_per_w // IDXW

    mesh = plsc.VectorSubcoreMesh(core_axis_name="c", subcore_axis_name="s")

    def body(x_hbm, table_hbm, out_hbm, idx_v, rows_v, idx_sem, row_sem):
        wid = lax.axis_index("s") * NC + lax.axis_index("c")
        row_base = wid * rows_per_w
        idx_base = wid * idxrows_per_w

        def step(i, carry):
            pltpu.async_copy(
                x_hbm.at[pl.ds(idx_base + i * KSUB, KSUB)], idx_v, idx_sem
            ).wait()
            for j in range(KSUB):
                pltpu.async_copy(
                    table_hbm.at[idx_v.at[j]],
                    rows_v.at[pl.ds(j * IDXW, IDXW)],
                    row_sem,
                ).wait()
            pltpu.sync_copy(rows_v, out_hbm.at[pl.ds(row_base + i * CHUNK, CHUNK)])
            return carry

        lax.fori_loop(0, n_iter, step, 0)

    return pl.kernel(
        body,
        out_type=jax.ShapeDtypeStruct((B, D), jnp.float32),
        mesh=mesh,
        scratch_types=[
            pltpu.VMEM((KSUB, IDXW), jnp.int32),
            pltpu.VMEM((CHUNK, D), jnp.float32),
            pltpu.SemaphoreType.DMA,
            pltpu.SemaphoreType.DMA,
        ],
        compiler_params=pltpu.CompilerParams(use_tc_tiling_on_sc=False),
    )


def kernel(x, table):
    B0, H = x.shape
    V, D = table.shape
    B = B0 * H
    xf = x.reshape(B // IDXW, IDXW)
    out = _build(B, V, D)(xf, table)
    return out.reshape(B0, H, D)


# speedup vs baseline: 1.1984x; 1.1984x over previous
import functools

import jax
import jax.numpy as jnp
from jax import lax
from jax.experimental import pallas as pl
from jax.experimental.pallas import tpu as pltpu
from jax.experimental.pallas import tpu_sc as plsc

NC = 2
NS = 16
NW = NC * NS

IDXW = 128
KSUB = 4
CHUNK = KSUB * IDXW


@functools.lru_cache(maxsize=None)
def _build(B, V, D):
    assert B % (NW * CHUNK) == 0
    rows_per_w = B // NW
    n_iter = rows_per_w // CHUNK
    idxrows_per_w = rows_per_w // IDXW
    assert n_iter >= 4 and (n_iter - 2) % 2 == 0

    mesh = plsc.VectorSubcoreMesh(core_axis_name="c", subcore_axis_name="s")

    def body(x_hbm, table_hbm, out_hbm,
             i0, i1, r0, r1, is0, is1, gs0, gs1, ss0, ss1):
        I = (i0, i1)
        R = (r0, r1)
        isem = (is0, is1)
        gsem = (gs0, gs1)
        ssem = (ss0, ss1)
        wid = lax.axis_index("s") * NC + lax.axis_index("c")
        row_base = wid * rows_per_w
        idx_base = wid * idxrows_per_w

        def start_idx(g, sl):
            pltpu.async_copy(
                x_hbm.at[pl.ds(idx_base + g * KSUB, KSUB)], I[sl], isem[sl])

        def wait_idx(sl):
            pltpu.make_async_copy(
                x_hbm.at[pl.ds(0, KSUB)], I[sl], isem[sl]).wait()

        def start_gathers(sl):
            for j in range(KSUB):
                pltpu.async_copy(
                    table_hbm.at[I[sl].at[j]],
                    R[sl].at[pl.ds(j * IDXW, IDXW)],
                    gsem[sl])

        def wait_gathers(sl):
            pltpu.make_async_copy(
                out_hbm.at[pl.ds(0, CHUNK)], R[sl], gsem[sl]).wait()

        def start_store(g, sl):
            pltpu.async_copy(
                R[sl], out_hbm.at[pl.ds(row_base + g * CHUNK, CHUNK)], ssem[sl])

        def wait_store(sl):
            pltpu.make_async_copy(
                R[sl], out_hbm.at[pl.ds(0, CHUNK)], ssem[sl]).wait()

        start_idx(0, 0)
        start_idx(1, 1)
        wait_idx(0)
        start_gathers(0)
        wait_idx(1)
        start_gathers(1)
        wait_gathers(0)
        start_store(0, 0)
        start_idx(2, 0)

        @pl.loop(1, n_iter - 1, step=2)
        def _steady(base):
            for b in range(2):
                g = base + b
                sl = (1 + b) % 2
                nsl = 1 - sl
                wait_store(nsl)
                wait_idx(nsl)
                start_gathers(nsl)
                wait_gathers(sl)
                start_store(g, sl)
                gnext = jnp.minimum(g + 2, n_iter - 1)
                start_idx(gnext, sl)

        fb = (n_iter - 1) % 2
        nfb = 1 - fb
        wait_store(nfb)
        wait_idx(nfb)
        wait_gathers(fb)
        start_store(n_iter - 1, fb)
        wait_store(fb)

    return pl.kernel(
        body,
        out_type=jax.ShapeDtypeStruct((B, D), jnp.float32),
        mesh=mesh,
        scratch_types=[
            pltpu.VMEM((KSUB, IDXW), jnp.int32),
            pltpu.VMEM((KSUB, IDXW), jnp.int32),
            pltpu.VMEM((CHUNK, D), jnp.float32),
            pltpu.VMEM((CHUNK, D), jnp.float32),
            pltpu.SemaphoreType.DMA,
            pltpu.SemaphoreType.DMA,
            pltpu.SemaphoreType.DMA,
            pltpu.SemaphoreType.DMA,
            pltpu.SemaphoreType.DMA,
            pltpu.SemaphoreType.DMA,
        ],
        compiler_params=pltpu.CompilerParams(use_tc_tiling_on_sc=False),
    )


def kernel(x, table):
    B0, H = x.shape
    V, D = table.shape
    B = B0 * H
    xf = x.reshape(B // IDXW, IDXW)
    out = _build(B, V, D)(xf, table)
    return out.reshape(B0, H, D)


# speedup vs baseline: 1.2036x; 1.0044x over previous
import functools

import jax
import jax.numpy as jnp
from jax import lax
from jax.experimental import pallas as pl
from jax.experimental.pallas import tpu as pltpu
from jax.experimental.pallas import tpu_sc as plsc

NC = 2
NS = 16
NW = NC * NS

BCH = 4
SPLITS = ((0, 96), (96, 104))


@functools.lru_cache(maxsize=None)
def _build(B0, H, V, D):
    assert B0 % (NW * BCH) == 0
    b_per_w = B0 // NW
    n_iter = b_per_w // BCH
    assert n_iter >= 4 and (n_iter - 2) % 2 == 0

    mesh = plsc.VectorSubcoreMesh(core_axis_name="c", subcore_axis_name="s")

    def body(x_hbm, table_hbm, out_hbm,
             i0, i1, r0, r1, is0, is1, gs0, gs1, ss0, ss1):
        I = (i0, i1)
        R = (r0, r1)
        isem = (is0, is1)
        gsem = (gs0, gs1)
        ssem = (ss0, ss1)
        wid = lax.axis_index("s") * NC + lax.axis_index("c")
        b_base = wid * b_per_w

        def start_idx(g, sl):
            pltpu.async_copy(
                x_hbm.at[pl.ds(b_base + g * BCH, BCH), :], I[sl], isem[sl])

        def wait_idx(sl):
            pltpu.make_async_copy(
                x_hbm.at[pl.ds(0, BCH), :], I[sl], isem[sl]).wait()

        def start_gathers(sl):
            for j in range(BCH):
                for (o, n) in SPLITS:
                    pltpu.async_copy(
                        table_hbm.at[I[sl].at[j, pl.ds(o, n)]],
                        R[sl].at[j, pl.ds(o, n)],
                        gsem[sl])

        def wait_gathers(sl):
            pltpu.make_async_copy(
                out_hbm.at[pl.ds(0, BCH), :, :], R[sl], gsem[sl]).wait()

        def start_store(g, sl):
            pltpu.async_copy(
                R[sl], out_hbm.at[pl.ds(b_base + g * BCH, BCH), :, :],
                ssem[sl])

        def wait_store(sl):
            pltpu.make_async_copy(
                R[sl], out_hbm.at[pl.ds(0, BCH), :, :], ssem[sl]).wait()

        start_idx(0, 0)
        start_idx(1, 1)
        wait_idx(0)
        start_gathers(0)
        wait_idx(1)
        start_gathers(1)
        wait_gathers(0)
        start_store(0, 0)
        start_idx(2, 0)

        @pl.loop(1, n_iter - 1, step=2)
        def _steady(base):
            for b in range(2):
                g = base + b
                sl = (1 + b) % 2
                nsl = 1 - sl
                wait_store(nsl)
                wait_idx(nsl)
                start_gathers(nsl)
                wait_gathers(sl)
                start_store(g, sl)
                gnext = jnp.minimum(g + 2, n_iter - 1)
                start_idx(gnext, sl)

        fb = (n_iter - 1) % 2
        nfb = 1 - fb
        wait_store(nfb)
        wait_idx(nfb)
        wait_gathers(fb)
        start_store(n_iter - 1, fb)
        wait_store(fb)

    return pl.kernel(
        body,
        out_type=jax.ShapeDtypeStruct((B0, H, D), jnp.float32),
        mesh=mesh,
        scratch_types=[
            pltpu.VMEM((BCH, H), jnp.int32),
            pltpu.VMEM((BCH, H), jnp.int32),
            pltpu.VMEM((BCH, H, D), jnp.float32),
            pltpu.VMEM((BCH, H, D), jnp.float32),
            pltpu.SemaphoreType.DMA,
            pltpu.SemaphoreType.DMA,
            pltpu.SemaphoreType.DMA,
            pltpu.SemaphoreType.DMA,
            pltpu.SemaphoreType.DMA,
            pltpu.SemaphoreType.DMA,
        ],
        compiler_params=pltpu.CompilerParams(use_tc_tiling_on_sc=False),
    )


def kernel(x, table):
    B0, H = x.shape
    V, D = table.shape
    return _build(B0, H, V, D)(x, table)
